# baseline (speedup 1.0000x reference)
import functools

import jax
import jax.numpy as jnp
from jax import lax
from jax.experimental import pallas as pl
from jax.experimental.pallas import tpu as pltpu
from jax.experimental.pallas import tpu_sc as plsc

NC = 2
NS = 16
NW = NC * NS
CH = 64


def _xw_body(x_ref, w_ref, o_ref):
    o_ref[...] = jnp.dot(x_ref[...], w_ref[0], preferred_element_type=jnp.float32)


def _finish_body(p_ref, b_ref, o_ref):
    o_ref[...] = jnp.maximum(p_ref[0] + p_ref[1] + b_ref[...], 0.0)


def kernel(x, edge_index, edge_type, W, b):
    n, d_in = x.shape
    r_rel, _, d_out = W.shape
    e = edge_index.shape[1]

    src = edge_index[0].astype(jnp.int32)
    dst = edge_index[1].astype(jnp.int32)
    etype = edge_type.astype(jnp.int32)

    nb = 10
    bm = n // nb
    xw = pl.pallas_call(
        _xw_body,
        grid=(r_rel, nb),
        in_specs=[
            pl.BlockSpec((bm, d_in), lambda r, i: (i, 0)),
            pl.BlockSpec((1, d_in, d_out), lambda r, i: (r, 0, 0)),
        ],
        out_specs=pl.BlockSpec((bm, d_out), lambda r, i: (r * nb + i, 0)),
        out_shape=jax.ShapeDtypeStruct((r_rel * n, d_out), jnp.float32),
    )(x, W)

    cpw = -(-e // (NW * CH))
    cpw = -(-cpw // 8) * 8
    e_pad = NW * CH * cpw
    pad = e_pad - e
    nq = cpw // 4
    src_p = jnp.pad(src, (0, pad)).reshape(NW, nq, 4, CH)
    et_p = jnp.pad(etype, (0, pad)).reshape(NW, nq, 4, CH)
    dst_p = jnp.pad(dst, (0, pad), constant_values=n).reshape(NW, nq, 4, CH)

    def _comb_body(s_ref, t_ref, d_ref, o_ref):
        o_ref[0, :, 0] = t_ref[0] * n + s_ref[0]
        o_ref[0, :, 1] = d_ref[0]

    idx_spec = pl.BlockSpec((1, nq, 4, CH), lambda i: (i, 0, 0, 0))
    comb = pl.pallas_call(
        _comb_body,
        grid=(NW,),
        in_specs=[idx_spec, idx_spec, idx_spec],
        out_specs=pl.BlockSpec((1, nq, 2, 4, CH), lambda i: (i, 0, 0, 0, 0)),
        out_shape=jax.ShapeDtypeStruct((NW, nq, 2, 4, CH), jnp.int32),
    )(src_p, et_p, dst_p)

    rows_per_tile = (-(-n // NS) + 7) // 8 * 8
    acc_rows = NS * rows_per_tile
    assert acc_rows > n
    zeros = jnp.zeros((rows_per_tile, d_out), jnp.float32)

    mesh = plsc.VectorSubcoreMesh(
        core_axis_name="c", subcore_axis_name="s", num_cores=NC, num_subcores=NS
    )

    @functools.partial(
        pl.kernel,
        out_type=jax.ShapeDtypeStruct((NC, acc_rows, d_out), jnp.float32),
        mesh=mesh,
        scratch_types=[
            pltpu.VMEM((2, 2, 4, CH), jnp.int32),
            [pltpu.VMEM((CH, d_out), jnp.float32)] * 4,
            pltpu.VMEM_SHARED((acc_rows, d_out), jnp.float32),
            [pltpu.SemaphoreType.DMA] * 4,
            [pltpu.SemaphoreType.DMA] * 4,
            [pltpu.SemaphoreType.DMA] * 2,
        ],
    )
    def _sc_agg(comb_hbm, xw_hbm, zeros_hbm, out_hbm,
                ring, bufs, acc, gsems, ssems, rsems):
        c = lax.axis_index("c")
        s = lax.axis_index("s")
        wid = s * NC + c
        pltpu.sync_copy(zeros_hbm, acc.at[pl.ds(s * rows_per_tile, rows_per_tile)])
        plsc.subcore_barrier()

        def stage(q, slot):
            pltpu.async_copy(comb_hbm.at[wid, q], ring.at[slot], rsems[slot])

        def wait_stage(slot):
            pltpu.make_async_copy(comb_hbm.at[wid, 0], ring.at[slot],
                                  rsems[slot]).wait()

        def run_quad(q, slot, nslot, prefetch, stage_next):
            for k in range(4):
                pltpu.make_async_copy(xw_hbm.at[ring.at[slot, 0, k]],
                                      bufs[k], gsems[k]).wait()
                pltpu.async_copy(bufs[k], acc.at[ring.at[slot, 1, k]],
                                 ssems[k], add=True)
            if prefetch:
                wait_stage(nslot)
            for k in range(4):
                pltpu.make_async_copy(bufs[k], acc.at[ring.at[slot, 1, k]],
                                      ssems[k]).wait()
                if prefetch:
                    pltpu.async_copy(xw_hbm.at[ring.at[nslot, 0, k]],
                                     bufs[k], gsems[k])
            if stage_next:
                stage(q + 2, slot)

        stage(0, 0)
        wait_stage(0)
        for k in range(4):
            pltpu.async_copy(xw_hbm.at[ring.at[0, 0, k]], bufs[k], gsems[k])
        stage(1, 1)

        def body(i, carry):
            q0 = i * 2
            run_quad(q0, 0, 1, True, True)
            run_quad(q0 + 1, 1, 0, True, True)
            return carry

        lax.fori_loop(0, nq // 2 - 1, body, 0)

        run_quad(nq - 2, 0, 1, True, False)
        run_quad(nq - 1, 1, 0, False, False)
        plsc.subcore_barrier()
        pltpu.sync_copy(
            acc.at[pl.ds(s * rows_per_tile, rows_per_tile)],
            out_hbm.at[c, pl.ds(s * rows_per_tile, rows_per_tile)],
        )

    partials = _sc_agg(comb, xw, zeros)

    out = pl.pallas_call(
        _finish_body,
        grid=(nb,),
        in_specs=[
            pl.BlockSpec((NC, bm, d_out), lambda i: (0, i, 0)),
            pl.BlockSpec((1, d_out), lambda i: (0, 0)),
        ],
        out_specs=pl.BlockSpec((bm, d_out), lambda i: (i, 0)),
        out_shape=jax.ShapeDtypeStruct((n, d_out), jnp.float32),
    )(partials, b.reshape(1, d_out))
    return out


# speedup vs baseline: 1.1848x; 1.1848x over previous
import functools

import jax
import jax.numpy as jnp
from jax import lax
from jax.experimental import pallas as pl
from jax.experimental.pallas import tpu as pltpu
from jax.experimental.pallas import tpu_sc as plsc

NC = 2
NS = 16
NW = NC * NS
CH = 128


def _xw_body(x_ref, w_ref, o_ref):
    o_ref[...] = jnp.dot(x_ref[...], w_ref[0], preferred_element_type=jnp.float32)


def _finish_body(p_ref, b_ref, o_ref):
    o_ref[...] = jnp.maximum(p_ref[0] + p_ref[1] + b_ref[...], 0.0)


def kernel(x, edge_index, edge_type, W, b):
    n, d_in = x.shape
    r_rel, _, d_out = W.shape
    e = edge_index.shape[1]

    src = edge_index[0].astype(jnp.int32)
    dst = edge_index[1].astype(jnp.int32)
    etype = edge_type.astype(jnp.int32)

    nb = 10
    bm = n // nb
    xw = pl.pallas_call(
        _xw_body,
        grid=(r_rel, nb),
        in_specs=[
            pl.BlockSpec((bm, d_in), lambda r, i: (i, 0)),
            pl.BlockSpec((1, d_in, d_out), lambda r, i: (r, 0, 0)),
        ],
        out_specs=pl.BlockSpec((bm, d_out), lambda r, i: (r * nb + i, 0)),
        out_shape=jax.ShapeDtypeStruct((r_rel * n, d_out), jnp.float32),
    )(x, W)

    cpw = -(-e // (NW * CH))
    cpw = -(-cpw // 4) * 4
    e_pad = NW * CH * cpw
    pad = e_pad - e
    src_p = jnp.pad(src, (0, pad)).reshape(e_pad // 128, 128)
    et_p = jnp.pad(etype, (0, pad)).reshape(e_pad // 128, 128)
    dst_p = jnp.pad(dst, (0, pad), constant_values=n)

    def _idx_body(s_ref, t_ref, o_ref):
        o_ref[...] = t_ref[...] * n + s_ref[...]

    rows_blk = e_pad // 128 // 16
    rowidx = pl.pallas_call(
        _idx_body,
        grid=(16,),
        in_specs=[
            pl.BlockSpec((rows_blk, 128), lambda i: (i, 0)),
            pl.BlockSpec((rows_blk, 128), lambda i: (i, 0)),
        ],
        out_specs=pl.BlockSpec((rows_blk, 128), lambda i: (i, 0)),
        out_shape=jax.ShapeDtypeStruct((e_pad // 128, 128), jnp.int32),
    )(src_p, et_p)

    nq = cpw // 2
    comb = jnp.stack(
        [rowidx.reshape(NW, nq, 2, CH), dst_p.reshape(NW, nq, 2, CH)], axis=2
    )

    rows_per_tile = (-(-n // NS) + 7) // 8 * 8
    acc_rows = NS * rows_per_tile
    assert acc_rows > n
    zeros = jnp.zeros((rows_per_tile, d_out), jnp.float32)

    mesh = plsc.VectorSubcoreMesh(
        core_axis_name="c", subcore_axis_name="s", num_cores=NC, num_subcores=NS
    )

    @functools.partial(
        pl.kernel,
        out_type=jax.ShapeDtypeStruct((NC, acc_rows, d_out), jnp.float32),
        mesh=mesh,
        scratch_types=[
            pltpu.VMEM((2, 2, 2, CH), jnp.int32),
            [pltpu.VMEM((CH, d_out), jnp.float32)] * 2,
            pltpu.VMEM_SHARED((acc_rows, d_out), jnp.float32),
            [pltpu.SemaphoreType.DMA] * 2,
            [pltpu.SemaphoreType.DMA] * 2,
            [pltpu.SemaphoreType.DMA] * 2,
        ],
    )
    def _sc_agg(comb_hbm, xw_hbm, zeros_hbm, out_hbm,
                ring, bufs, acc, gsems, ssems, rsems):
        c = lax.axis_index("c")
        s = lax.axis_index("s")
        wid = s * NC + c
        pltpu.sync_copy(zeros_hbm, acc.at[pl.ds(s * rows_per_tile, rows_per_tile)])
        plsc.subcore_barrier()

        def stage(q, slot):
            pltpu.async_copy(comb_hbm.at[wid, q], ring.at[slot], rsems[slot])

        def wait_stage(slot):
            pltpu.make_async_copy(comb_hbm.at[wid, 0], ring.at[slot],
                                  rsems[slot]).wait()

        def run_quad(q, slot, nslot, prefetch, stage_next):
            for k in range(2):
                pltpu.make_async_copy(xw_hbm.at[ring.at[slot, 0, k]],
                                      bufs[k], gsems[k]).wait()
                pltpu.async_copy(bufs[k], acc.at[ring.at[slot, 1, k]],
                                 ssems[k], add=True)
            if prefetch:
                wait_stage(nslot)
            for k in range(2):
                pltpu.make_async_copy(bufs[k], acc.at[ring.at[slot, 1, k]],
                                      ssems[k]).wait()
                if prefetch:
                    pltpu.async_copy(xw_hbm.at[ring.at[nslot, 0, k]],
                                     bufs[k], gsems[k])
            if stage_next:
                stage(q + 2, slot)

        stage(0, 0)
        wait_stage(0)
        for k in range(2):
            pltpu.async_copy(xw_hbm.at[ring.at[0, 0, k]], bufs[k], gsems[k])
        stage(1, 1)

        def body(i, carry):
            q0 = i * 2
            run_quad(q0, 0, 1, True, True)
            run_quad(q0 + 1, 1, 0, True, True)
            return carry

        lax.fori_loop(0, nq // 2 - 1, body, 0)

        run_quad(nq - 2, 0, 1, True, False)
        run_quad(nq - 1, 1, 0, False, False)
        plsc.subcore_barrier()
        pltpu.sync_copy(
            acc.at[pl.ds(s * rows_per_tile, rows_per_tile)],
            out_hbm.at[c, pl.ds(s * rows_per_tile, rows_per_tile)],
        )

    partials = _sc_agg(comb, xw, zeros)

    out = pl.pallas_call(
        _finish_body,
        grid=(nb,),
        in_specs=[
            pl.BlockSpec((NC, bm, d_out), lambda i: (0, i, 0)),
            pl.BlockSpec((1, d_out), lambda i: (0, 0)),
        ],
        out_specs=pl.BlockSpec((bm, d_out), lambda i: (i, 0)),
        out_shape=jax.ShapeDtypeStruct((n, d_out), jnp.float32),
    )(partials, b.reshape(1, d_out))
    return out
